# baseline (speedup 1.0000x reference)
import functools

import jax
import jax.numpy as jnp
import numpy as np
from jax import lax
from jax.experimental import pallas as pl
from jax.experimental.pallas import tpu as pltpu
from jax.experimental.pallas import tpu_sc as plsc

_LANES = 16
_NW = 32
_IDX_CHUNK = 128


def _make_gather(n_tok, vocab, dim, seq):
    per_w = n_tok // _NW
    n_chunks = per_w // _IDX_CHUNK
    n_groups = per_w // _LANES
    mod = vocab - 1
    mdim = 2 * dim
    mesh = plsc.VectorSubcoreMesh(core_axis_name="c", subcore_axis_name="s")

    @functools.partial(
        pl.kernel,
        mesh=mesh,
        out_type=jax.ShapeDtypeStruct((n_tok, dim), jnp.float32),
        scratch_types=[
            pltpu.VMEM((per_w,), jnp.int32),
            pltpu.VMEM((per_w,), jnp.int32),
            pltpu.VMEM((n_chunks, _IDX_CHUNK), jnp.int32),
            pltpu.VMEM((per_w,), jnp.int32),
            pltpu.VMEM((_IDX_CHUNK, mdim), jnp.float32),
            pltpu.VMEM((_IDX_CHUNK, mdim), jnp.float32),
            pltpu.VMEM((per_w, dim), jnp.float32),
            pltpu.SemaphoreType.DMA,
            pltpu.SemaphoreType.DMA,
        ],
        compiler_params=pltpu.CompilerParams(use_tc_tiling_on_sc=True,
                                             needs_layout_passes=False),
    )
    def gather_kernel(tok_hbm, tokp_hbm, table128_hbm, h_hbm,
                      cur_v, prev_v, idx_m, par_v, g0_v, g1_v, h_v,
                      sem0, sem1):
        wid = lax.axis_index("s") * 2 + lax.axis_index("c")
        base = wid * per_w
        pltpu.sync_copy(tok_hbm.at[pl.ds(base, per_w)], cur_v)
        pltpu.sync_copy(tokp_hbm.at[pl.ds(base, per_w)], prev_v)
        modv = jnp.full((_LANES,), mod, dtype=jnp.int32)
        for i in range(n_groups):
            cur = cur_v[pl.ds(i * _LANES, _LANES)]
            prev = prev_v[pl.ds(i * _LANES, _LANES)]
            h = (cur * 36313) ^ (prev * 27191)
            h = lax.rem(h, modv)
            pos = base + i * _LANES + lax.iota(jnp.int32, _LANES)
            h = jnp.where((pos & (seq - 1)) == 0, mod, h)
            idx_m[(i * _LANES) // _IDX_CHUNK,
                  pl.ds((i * _LANES) % _IDX_CHUNK, _LANES)] = h >> 1
            par_v[pl.ds(i * _LANES, _LANES)] = (h & 1) * dim

        bufs = (g0_v, g1_v)
        sems = (sem0, sem1)
        iota = lax.iota(jnp.int32, _LANES)
        grp_per_chunk = _IDX_CHUNK // _LANES

        def fire(j):
            return pltpu.async_copy(
                table128_hbm.at[idx_m.at[jnp.int32(j)]],
                bufs[j % 2], sems[j % 2])

        def make_select(j):
            buf = bufs[j % 2]

            def select(grp, _):
                lrows = grp * _LANES + iota
                grows = j * _IDX_CHUNK + lrows
                pcols = plsc.load_gather(par_v, [grows])
                for c in range(dim):
                    vals = plsc.load_gather(buf, [lrows, pcols + c])
                    plsc.store_scatter(
                        h_v, [grows, jnp.full((_LANES,), c, jnp.int32)], vals)
                return ()

            return select

        pend = fire(0)
        for j in range(n_chunks):
            nxt = fire(j + 1) if j + 1 < n_chunks else None
            pend.wait()
            lax.fori_loop(jnp.int32(0), jnp.int32(grp_per_chunk),
                          make_select(j), ())
            pend = nxt
        pltpu.sync_copy(h_v, h_hbm.at[pl.ds(base, per_w)])

    return gather_kernel


_ZERO = np.int32(0)


def _mm_body(h_ref, w_ref, scale_ref, o_ref):
    o_ref[...] = lax.dot_general(
        h_ref[...], w_ref[...],
        dimension_numbers=(((1,), (1,)), ((), ())),
        preferred_element_type=jnp.float32) * scale_ref[0]


def _make_matmul(n_tok, dim, model_dim, block_rows=1024):
    grid = n_tok // block_rows
    return pl.pallas_call(
        _mm_body,
        grid=(grid,),
        in_specs=[
            pl.BlockSpec((block_rows, dim), lambda i: (i, _ZERO)),
            pl.BlockSpec((model_dim, dim), lambda i: (_ZERO, _ZERO)),
            pl.BlockSpec((1,), lambda i: (_ZERO,), memory_space=pltpu.SMEM),
        ],
        out_specs=pl.BlockSpec((block_rows, model_dim), lambda i: (i, _ZERO)),
        out_shape=jax.ShapeDtypeStruct((n_tok, model_dim), jnp.float32),
    )


def kernel(token_ids, table, W_proj, scale):
    batch, seq = token_ids.shape
    vocab, dim = table.shape
    model_dim = W_proj.shape[0]
    n_tok = batch * seq

    tok = token_ids.astype(jnp.int32).reshape(-1)
    tok_prev = jnp.concatenate([jnp.zeros((1,), jnp.int32), tok[:-1]])
    table128 = table.reshape(vocab // 2, 2 * dim)

    h = _make_gather(n_tok, vocab, dim, seq)(tok, tok_prev, table128)
    scale1 = jnp.reshape(scale, (1,)).astype(jnp.float32)
    out = _make_matmul(n_tok, dim, model_dim)(h, W_proj, scale1)
    return out.reshape(batch, seq, model_dim)


# speedup vs baseline: 1.7901x; 1.7901x over previous
import functools

import jax
import jax.numpy as jnp
import numpy as np
from jax import lax
from jax.experimental import pallas as pl
from jax.experimental.pallas import tpu as pltpu
from jax.experimental.pallas import tpu_sc as plsc

_LANES = 16
_NW = 32
_SLAB = 512
_TCHUNK = 2048


def _make_gather(n_tok, vocab, dim, seq):
    mod = vocab - 1
    n_slabs = (vocab + _SLAB - 1) // _SLAB
    last_w = vocab - (n_slabs - 1) * _SLAB
    spw = (n_slabs + _NW - 1) // _NW
    spw = n_slabs // _NW
    w31_slabs = n_slabs - (_NW - 1) * spw
    max_spw = max(spw, w31_slabs)
    n_out = n_tok + _LANES
    mdim = 2 * dim
    mesh = plsc.VectorSubcoreMesh(core_axis_name="c", subcore_axis_name="s")

    @functools.partial(
        pl.kernel,
        mesh=mesh,
        out_type=jax.ShapeDtypeStruct((n_out, mdim), jnp.float32),
        scratch_types=[
            pltpu.VMEM((_TCHUNK,), jnp.int32),
            pltpu.VMEM((_TCHUNK,), jnp.int32),
            pltpu.VMEM((n_tok,), jnp.int32),
            pltpu.VMEM((n_tok,), jnp.int32),
            pltpu.VMEM((dim, _SLAB), jnp.float32),
            pltpu.VMEM((dim, 64), jnp.float32),
            pltpu.VMEM((_LANES, mdim), jnp.float32),
            pltpu.VMEM((64,), jnp.int32),
            pltpu.VMEM((64,), jnp.int32),
            pltpu.VMEM((64,), jnp.int32),
            pltpu.VMEM((_LANES,), jnp.int32),
            pltpu.VMEM((_LANES,), jnp.int32),
            pltpu.VMEM((_LANES,), jnp.int32),
            pltpu.SemaphoreType.DMA,
        ],
        compiler_params=pltpu.CompilerParams(use_tc_tiling_on_sc=True,
                                             needs_layout_passes=False),
    )
    def gather_kernel(tok_hbm, tokp_hbm, tableT_hbm, tlast_hbm, h2_hbm,
                      tokc_v, tokp_v, comp_v, buck_v, slab_v, mini_v, src_v,
                      counts_v, offs_v, cursor_v, tmps_v, tmpv_v, tmpm_v,
                      sem):
        wid = lax.axis_index("s") * 2 + lax.axis_index("c")
        start_slab = wid * spw
        is_last = wid == (_NW - 1)
        r_lo = start_slab * _SLAB
        r_hi = jnp.where(is_last, n_slabs * _SLAB, r_lo + spw * _SLAB)
        iota = lax.iota(jnp.int32, _LANES)
        zi = jnp.zeros((_LANES,), jnp.int32)
        zf = jnp.zeros((_LANES,), jnp.float32)
        ones = jnp.ones((_LANES,), jnp.int32)
        lane0 = iota == 0
        modv = jnp.full((_LANES,), mod, dtype=jnp.int32)

        for b in range(64 // _LANES):
            counts_v[pl.ds(b * _LANES, _LANES)] = zi
        for rr in range(_LANES):
            for cc in range(dim // _LANES):
                src_v[rr, pl.ds(dim + cc * _LANES, _LANES)] = zf

        def chunk_body(ch, cnt):
            pltpu.sync_copy(tok_hbm.at[pl.ds(ch * _TCHUNK, _TCHUNK)], tokc_v)
            pltpu.sync_copy(tokp_hbm.at[pl.ds(ch * _TCHUNK, _TCHUNK)], tokp_v)

            def grp(i, cnt):
                cur = plsc.load_gather(tokc_v, [i * _LANES + iota])
                prev = plsc.load_gather(tokp_v, [i * _LANES + iota])
                h = (cur * 36313) ^ (prev * 27191)
                h = lax.rem(h, modv)
                pos = ch * _TCHUNK + i * _LANES + iota
                h = jnp.where((pos & (seq - 1)) == 0, mod, h)
                m = (h >= r_lo) & (h < r_hi)
                packed = ((h - r_lo) << 15) | pos
                plsc.store_compressed(comp_v.at[pl.ds(cnt, _LANES)], packed,
                                      mask=m)
                return cnt + jnp.sum(m.astype(jnp.int32), dtype=jnp.int32)

            return lax.fori_loop(jnp.int32(0), jnp.int32(_TCHUNK // _LANES),
                                 grp, cnt)

        n_local = lax.fori_loop(jnp.int32(0), jnp.int32(n_tok // _TCHUNK),
                                chunk_body, jnp.int32(0))

        def cb(g, _):
            lid = g * _LANES + iota
            m = lid < n_local
            v = plsc.load_gather(comp_v, [jnp.where(m, lid, 0)])
            s = (v >> 24) & 63
            plsc.addupdate_scatter(counts_v, [s], ones, mask=m)
            return ()

        lax.fori_loop(jnp.int32(0), (n_local + _LANES - 1) >> 4, cb, ())

        carry = jnp.int32(0)
        for b in range(64 // _LANES):
            c = counts_v[pl.ds(b * _LANES, _LANES)]
            cs = plsc.cumsum(c)
            offs_v[pl.ds(b * _LANES, _LANES)] = cs - c + carry
            carry = carry + jnp.sum(c, dtype=jnp.int32)
        for b in range(64 // _LANES):
            cursor_v[pl.ds(b * _LANES, _LANES)] = offs_v[pl.ds(b * _LANES,
                                                              _LANES)]

        def pgrp(g, _):
            lid = g * _LANES + iota
            m = lid < n_local
            v = plsc.load_gather(comp_v, [jnp.where(m, lid, 0)])
            tmps_v[pl.ds(0, _LANES)] = (v >> 24) & 63
            tmpv_v[pl.ds(0, _LANES)] = v
            tmpm_v[pl.ds(0, _LANES)] = m.astype(jnp.int32)
            for l in range(_LANES):
                li = jnp.full((_LANES,), l, jnp.int32)
                sl_ = plsc.load_gather(tmps_v, [li])
                vl = plsc.load_gather(tmpv_v, [li])
                ml = plsc.load_gather(tmpm_v, [li])
                wm = lane0 & (ml > 0)
                p = plsc.load_gather(cursor_v, [sl_])
                plsc.store_scatter(buck_v, [p], vl, mask=wm)
                plsc.store_scatter(cursor_v, [sl_], p + 1, mask=wm)
            return ()

        lax.fori_loop(jnp.int32(0), (n_local + _LANES - 1) >> 4, pgrp, ())

        def extract_bucket(sl, src_ref):
            slv = jnp.full((_LANES,), sl, jnp.int32)
            off = plsc.load_gather(offs_v, [slv])
            cntb = plsc.load_gather(counts_v, [slv])
            cnt_sc = jnp.max(cntb)

            def egrp(g, _):
                lid = off + g * _LANES + iota
                m = (g * _LANES + iota) < cntb
                v = plsc.load_gather(buck_v, [jnp.where(m, lid, 0)])
                v = jnp.where(m, v, 0)
                col = jnp.where(m, (v >> 15) - sl * _SLAB, 0)
                tsc = jnp.where(m, v & 32767, n_tok + iota)
                for f in range(dim):
                    fv = jnp.full((_LANES,), f, jnp.int32)
                    vals = plsc.load_gather(src_ref, [fv, col])
                    plsc.store_scatter(src_v, [iota, fv], vals)
                pltpu.async_copy(src_v, h2_hbm.at[tsc], sem).wait()
                return ()

            lax.fori_loop(jnp.int32(0), (cnt_sc + _LANES - 1) >> 4, egrp, ())

        n_full = jnp.where(is_last, jnp.int32(w31_slabs - 1), jnp.int32(spw))

        def slab_body(sl, _):
            pltpu.sync_copy(
                tableT_hbm.at[:, pl.ds((start_slab + sl) * _SLAB, _SLAB)],
                slab_v)
            extract_bucket(sl, slab_v)
            return ()

        lax.fori_loop(jnp.int32(0), n_full, slab_body, ())

        @pl.when(is_last)
        def _():
            pltpu.sync_copy(tlast_hbm, mini_v)
            extract_bucket(jnp.int32(w31_slabs - 1), mini_v)

    return gather_kernel


_ZERO = np.int32(0)


def _mm_body(h_ref, w_ref, scale_ref, o_ref):
    o_ref[...] = lax.dot_general(
        h_ref[...][:, :w_ref.shape[1]], w_ref[...],
        dimension_numbers=(((1,), (1,)), ((), ())),
        preferred_element_type=jnp.float32) * scale_ref[0]


def _make_matmul(n_tok, mdim, model_dim, block_rows=1024):
    grid = n_tok // block_rows
    return pl.pallas_call(
        _mm_body,
        grid=(grid,),
        in_specs=[
            pl.BlockSpec((block_rows, mdim), lambda i: (i, _ZERO)),
            pl.BlockSpec((model_dim, mdim // 2), lambda i: (_ZERO, _ZERO)),
            pl.BlockSpec((1,), lambda i: (_ZERO,), memory_space=pltpu.SMEM),
        ],
        out_specs=pl.BlockSpec((block_rows, model_dim), lambda i: (i, _ZERO)),
        out_shape=jax.ShapeDtypeStruct((n_tok, model_dim), jnp.float32),
    )


def kernel(token_ids, table, W_proj, scale):
    batch, seq = token_ids.shape
    vocab, dim = table.shape
    model_dim = W_proj.shape[0]
    n_tok = batch * seq

    tok = token_ids.astype(jnp.int32).reshape(-1)
    tok_prev = jnp.concatenate([jnp.zeros((1,), jnp.int32), tok[:-1]])
    tableT = table.T
    n_slabs = (vocab + _SLAB - 1) // _SLAB
    last_w = vocab - (n_slabs - 1) * _SLAB
    tlast = tableT[:, (n_slabs - 1) * _SLAB:]

    h2 = _make_gather(n_tok, vocab, dim, seq)(tok, tok_prev, tableT, tlast)
    scale1 = jnp.reshape(scale, (1,)).astype(jnp.float32)
    out = _make_matmul(n_tok, 2 * dim, model_dim)(h2[:n_tok], W_proj, scale1)
    return out.reshape(batch, seq, model_dim)
